# baseline (speedup 1.0000x reference)
import functools

import jax
import jax.numpy as jnp
from jax import lax
from jax.experimental import pallas as pl
from jax.experimental.pallas import tpu as pltpu
from jax.experimental.pallas import tpu_sc as plsc

B = 16384
D = 64
NF = 6
NP = 3
PW = 2 * D
NC, NS = 2, 16
NW = NC * NS
R = B // NW
C = 128
NCH = R // C
L = 16

PAIR_VB = (24, 100, 31)


def kernel(dayofweek, time, sex, age, month, day,
           W_dayofweek, W_time, W_sex, W_age, W_month, W_day):
    def pair_table(Wa, Wb):
        va, vb = Wa.shape[0], Wb.shape[0]
        return jnp.concatenate(
            [jnp.repeat(Wa, vb, axis=0), jnp.tile(Wb, (va, 1))], axis=1)

    T0 = pair_table(W_dayofweek, W_time)
    T1 = pair_table(W_sex, W_age)
    T2 = pair_table(W_month, W_day)

    mesh = plsc.VectorSubcoreMesh(
        core_axis_name="c", subcore_axis_name="s",
        num_cores=NC, num_subcores=NS)

    @functools.partial(
        pl.kernel,
        out_type=jax.ShapeDtypeStruct((B, NF * D), jnp.float32),
        mesh=mesh,
        scratch_types=[
            pltpu.VMEM((NF * R,), jnp.int32),
            pltpu.VMEM((NP * R,), jnp.int32),
            pltpu.VMEM((NP, 2, C, PW), jnp.float32),
        ] + [pltpu.SemaphoreType.DMA] * 12,
    )
    def sck(i0, i1, i2, i3, i4, i5, t0, t1, t2,
            out, raw_v, cidx_v, bufs, *sems):
        wid = lax.axis_index("s") * NC + lax.axis_index("c")
        base = wid * R
        idxs = (i0, i1, i2, i3, i4, i5)
        tables = (t0, t1, t2)
        gsem = [[sems[p * 2 + s] for s in range(2)] for p in range(NP)]
        wsem = [[sems[6 + p * 2 + s] for s in range(2)] for p in range(NP)]
        stage = [pltpu.async_copy(idxs[f].at[pl.ds(base, R)],
                                  raw_v.at[pl.ds(f * R, R)], gsem[0][0])
                 for f in range(NF)]
        for cp in stage:
            cp.wait()
        for p in range(NP):
            vb = PAIR_VB[p]
            for j in range(R // L):
                ia = raw_v[pl.ds((2 * p) * R + j * L, L)]
                ib = raw_v[pl.ds((2 * p + 1) * R + j * L, L)]
                cidx_v[pl.ds(p * R + j * L, L)] = ia * vb + ib

        gath = [[None, None] for _ in range(NP)]
        writes = [[None] * NCH for _ in range(NP)]
        for c in range(NCH + 1):
            if c < NCH:
                s = c % 2
                for p in range(NP):
                    if c >= 2:
                        writes[p][c - 2].wait()
                    gath[p][s] = pltpu.async_copy(
                        tables[p].at[cidx_v.at[pl.ds(p * R + c * C, C)]],
                        bufs.at[p, s], gsem[p][s])
            if c >= 1:
                s1 = (c - 1) % 2
                for p in range(NP):
                    gath[p][s1].wait()
                    writes[p][c - 1] = pltpu.async_copy(
                        bufs.at[p, s1],
                        out.at[pl.ds(base + (c - 1) * C, C),
                               pl.ds(p * PW, PW)],
                        wsem[p][s1])
        for p in range(NP):
            writes[p][NCH - 2].wait()
            writes[p][NCH - 1].wait()

    return sck(dayofweek.astype(jnp.int32), time.astype(jnp.int32),
               sex.astype(jnp.int32), age.astype(jnp.int32),
               month.astype(jnp.int32), day.astype(jnp.int32),
               T0, T1, T2)


# speedup vs baseline: 1.0068x; 1.0068x over previous
import functools

import jax
import jax.numpy as jnp
from jax import lax
from jax.experimental import pallas as pl
from jax.experimental.pallas import tpu as pltpu
from jax.experimental.pallas import tpu_sc as plsc

B = 16384
D = 64
NF = 6
NP = 3
PW = 2 * D
NC, NS = 2, 16
NW = NC * NS
R = B // NW
C = 128
NCH = R // C
L = 16

PAIR_VB = (24, 100, 31)


def kernel(dayofweek, time, sex, age, month, day,
           W_dayofweek, W_time, W_sex, W_age, W_month, W_day):
    def pair_table(Wa, Wb):
        va, vb = Wa.shape[0], Wb.shape[0]
        return jnp.concatenate(
            [jnp.repeat(Wa, vb, axis=0), jnp.tile(Wb, (va, 1))], axis=1)

    T0 = pair_table(W_dayofweek, W_time)
    T1 = pair_table(W_sex, W_age)
    T2 = pair_table(W_month, W_day)

    mesh = plsc.VectorSubcoreMesh(
        core_axis_name="c", subcore_axis_name="s",
        num_cores=NC, num_subcores=NS)

    @functools.partial(
        pl.kernel,
        out_type=jax.ShapeDtypeStruct((B, NF * D), jnp.float32),
        mesh=mesh,
        scratch_types=[
            pltpu.VMEM((NF * R,), jnp.int32),
            pltpu.VMEM((NP * R,), jnp.int32),
            pltpu.VMEM((2, C, NF * D), jnp.float32),
        ] + [pltpu.SemaphoreType.DMA] * 7,
    )
    def sck(i0, i1, i2, i3, i4, i5, t0, t1, t2,
            out, raw_v, cidx_v, asm_v, g0, g1, w0, w1, s0, s1, s2):
        wid = lax.axis_index("s") * NC + lax.axis_index("c")
        base = wid * R
        idxs = (i0, i1, i2, i3, i4, i5)
        tables = (t0, t1, t2)
        gsem = (g0, g1)
        wsem = (w0, w1)
        ssem = (s0, s1, s2)

        stage = [[pltpu.async_copy(idxs[2 * p + h].at[pl.ds(base, R)],
                                   raw_v.at[pl.ds((2 * p + h) * R, R)],
                                   ssem[p])
                  for h in range(2)] for p in range(NP)]

        def compute_cidx(p, j):
            ia = raw_v[pl.ds((2 * p) * R + j * L, L)]
            ib = raw_v[pl.ds((2 * p + 1) * R + j * L, L)]
            cidx_v[pl.ds(p * R + j * L, L)] = ia * PAIR_VB[p] + ib

        def fire(p, c, s):
            return pltpu.async_copy(
                tables[p].at[cidx_v.at[pl.ds(p * R + c * C, C)]],
                asm_v.at[s, :, pl.ds(p * PW, PW)], gsem[s])

        gath = [[], []]
        for p in range(NP):
            for cp in stage[p]:
                cp.wait()
            for j in range(2 * C // L):
                compute_cidx(p, j)
            gath[0].append(fire(p, 0, 0))
            gath[1].append(fire(p, 1, 1))
        for p in range(NP):
            for j in range(2 * C // L, R // L):
                compute_cidx(p, j)

        writes = [None] * NCH
        for c in range(NCH):
            s = c % 2
            for cp in gath[s]:
                cp.wait()
            writes[c] = pltpu.async_copy(
                asm_v.at[s], out.at[pl.ds(base + c * C, C), :], wsem[s])
            if c + 2 < NCH:
                writes[c].wait()
                gath[s] = [fire(p, c + 2, s) for p in range(NP)]
        writes[NCH - 2].wait()
        writes[NCH - 1].wait()

    return sck(dayofweek.astype(jnp.int32), time.astype(jnp.int32),
               sex.astype(jnp.int32), age.astype(jnp.int32),
               month.astype(jnp.int32), day.astype(jnp.int32),
               T0, T1, T2)
